# Initial kernel scaffold; baseline (speedup 1.0000x reference)
import jax
import jax.numpy as jnp
from jax.experimental import pallas as pl


def kernel(x, edge_index, W1l, W1r, b1, W2l, W2r, b2):
    raise NotImplementedError("write your pallas kernel here")


# speedup vs baseline: 5.2667x; 5.2667x over previous
import functools

import jax
import jax.numpy as jnp
from jax import lax
from jax.experimental import pallas as pl
from jax.experimental.pallas import tpu as pltpu
from jax.experimental.pallas import tpu_sc as plsc

NC = 2
NS = 16
CH = 80
CNTW = 16
DH = 64


def _sc_pass_body(NP, NCH, refs):
    (t_h, src_h, dst_h, zacc_h, zcnt_h, ones_h,
     acc_out, cnt_out, sidx, didx, rows, onesv, acc_s, cnt_s) = refs

    c = lax.axis_index("c")
    s = lax.axis_index("s")
    rpw = NP // NS
    rbase = s * rpw

    pltpu.sync_copy(zacc_h.at[pl.ds(rbase, rpw)], acc_s.at[pl.ds(rbase, rpw)])
    pltpu.sync_copy(zcnt_h.at[pl.ds(rbase, rpw)], cnt_s.at[pl.ds(rbase, rpw)])
    pltpu.sync_copy(ones_h, onesv)
    pltpu.sync_copy(src_h.at[s], sidx)
    pltpu.sync_copy(dst_h.at[s], didx)
    plsc.subcore_barrier()

    @pl.loop(0, NCH)
    def _(j):
        pltpu.sync_copy(t_h.at[c].at[sidx.at[j]], rows)
        pltpu.sync_copy(rows, acc_s.at[didx.at[j]], add=True)
        pltpu.sync_copy(onesv, cnt_s.at[didx.at[j]], add=True)

    plsc.subcore_barrier()
    pltpu.sync_copy(acc_s.at[pl.ds(rbase, rpw)], acc_out.at[c, pl.ds(rbase, rpw)])
    pltpu.sync_copy(cnt_s.at[pl.ds(rbase, rpw)], cnt_out.at[c, pl.ds(rbase, rpw)])


def _make_sc_pass(NP, E):
    EPC = E // NS
    NCH = EPC // CH
    mesh = plsc.VectorSubcoreMesh(core_axis_name="c", subcore_axis_name="s")
    out_type = [
        jax.ShapeDtypeStruct((NC, NP, DH), jnp.float32),
        jax.ShapeDtypeStruct((NC, NP, CNTW), jnp.float32),
    ]
    scratch = [
        pltpu.VMEM((NCH, CH), jnp.int32),
        pltpu.VMEM((NCH, CH), jnp.int32),
        pltpu.VMEM((CH, DH), jnp.float32),
        pltpu.VMEM((CH, CNTW), jnp.float32),
        pltpu.VMEM_SHARED((NP, DH), jnp.float32),
        pltpu.VMEM_SHARED((NP, CNTW), jnp.float32),
    ]

    def body(*refs):
        _sc_pass_body(NP, NCH, refs)

    return pl.kernel(body, out_type=out_type, mesh=mesh, scratch_types=scratch,
                     compiler_params=pltpu.CompilerParams(use_tc_tiling_on_sc=False))


def _combine_body(acc_ref, cnt_ref, t_ref, wl_ref, wr_ref, b_ref, o_ref, *,
                  relu, split_out):
    s = jnp.concatenate([acc_ref[0], acc_ref[1]], axis=1)
    t = jnp.concatenate([t_ref[0], t_ref[1]], axis=1)
    c = cnt_ref[0, :, 0:1]
    agg = s / jnp.maximum(c, 1.0)
    r = (jnp.dot(agg, wl_ref[...], preferred_element_type=jnp.float32)
         + jnp.dot(t, wr_ref[...], preferred_element_type=jnp.float32)
         + b_ref[...])
    if relu:
        r = jnp.maximum(r, 0.0)
    if split_out:
        o_ref[0] = r[:, :DH]
        o_ref[1] = r[:, DH:]
    else:
        o_ref[...] = r


def _make_combine(relu, split_out, N, NP, D, BN=1000):
    grid = (N // BN,)
    if split_out:
        out_spec = pl.BlockSpec((NC, BN, DH), lambda i: (0, i, 0))
        out_shape = jax.ShapeDtypeStruct((NC, N, DH), jnp.float32)
    else:
        out_spec = pl.BlockSpec((BN, D), lambda i: (i, 0))
        out_shape = jax.ShapeDtypeStruct((N, D), jnp.float32)
    return pl.pallas_call(
        functools.partial(_combine_body, relu=relu, split_out=split_out),
        grid=grid,
        in_specs=[
            pl.BlockSpec((NC, BN, DH), lambda i: (0, i, 0)),
            pl.BlockSpec((1, BN, CNTW), lambda i: (0, i, 0)),
            pl.BlockSpec((NC, BN, DH), lambda i: (0, i, 0)),
            pl.BlockSpec((D, D), lambda i: (0, 0)),
            pl.BlockSpec((D, D), lambda i: (0, 0)),
            pl.BlockSpec((1, D), lambda i: (0, 0)),
        ],
        out_specs=out_spec,
        out_shape=out_shape,
    )


def kernel(x, edge_index, W1l, W1r, b1, W2l, W2r, b2):
    N, D = x.shape
    E = edge_index.shape[1]
    NP = ((N + 8 * NS - 1) // (8 * NS)) * (8 * NS)
    EPC = E // NS
    NCH = EPC // CH
    src = edge_index[0].reshape(NS, NCH, CH)
    dst = edge_index[1].reshape(NS, NCH, CH)
    zacc = jnp.zeros((NP, DH), jnp.float32)
    zcnt = jnp.zeros((NP, CNTW), jnp.float32)
    ones = jnp.ones((CH, CNTW), jnp.float32)
    x2 = jnp.stack([x[:, :DH], x[:, DH:]], axis=0)

    sc = _make_sc_pass(NP, E)
    comb1 = _make_combine(True, True, N, NP, D)
    comb2 = _make_combine(False, False, N, NP, D)

    acc1, cnt1 = sc(x2, src, dst, zacc, zcnt, ones)
    h2 = comb1(acc1, cnt1[:1], x2, W1l, W1r, b1.reshape(1, D))
    acc2, _ = sc(h2, src, dst, zacc, zcnt, ones)
    out = comb2(acc2, cnt1[:1], h2, W2l, W2r, b2.reshape(1, D))
    return out
